# baseline (speedup 1.0000x reference)
import dataclasses
import functools

import jax
import jax.numpy as jnp
from jax.experimental import pallas as pl
from jax.experimental.pallas import tpu as pltpu
from jax.experimental.pallas import tpu_sc as plsc

N = 10000
E = 160000
IN_FEATS = 256
HEADS = 4

NC, NS, L = 2, 16, 16
NPAD = 10240
TPT = NPAD // NS
HALF = NPAD // 2
HPT = HALF // NS
BN = 1000
NT = N // BN

EW = 400
ET1 = E // NS
NW1 = ET1 // EW
MW = 80
NMW = ET1 // MW

_MESH = plsc.VectorSubcoreMesh(core_axis_name="c", subcore_axis_name="s")
_CP = pltpu.CompilerParams()
if "needs_layout_passes" in pltpu.CompilerParams.__dataclass_fields__:
    _CP = dataclasses.replace(_CP, needs_layout_passes=False)


def _k1_body(x_ref, w_ref, p_ref, feat_ref, elr0_ref, elr1_ref):
    c = pl.program_id(1)
    fc = jnp.dot(x_ref[...], w_ref[0], preferred_element_type=jnp.float32, precision=jax.lax.Precision.HIGHEST)
    feat_ref[0] = fc
    pe = jnp.dot(fc, p_ref[0], preferred_element_type=jnp.float32, precision=jax.lax.Precision.HIGHEST)

    @pl.when(c == 0)
    def _():
        elr0_ref[...] = pe[:, :4]
        elr1_ref[...] = pe[:, 4:]

    @pl.when(c != 0)
    def _():
        elr0_ref[...] += pe[:, :4]
        elr1_ref[...] += pe[:, 4:]


def _k1(features, W1r, P1):
    return pl.pallas_call(
        _k1_body,
        grid=(NT, 8),
        in_specs=[
            pl.BlockSpec((BN, IN_FEATS), lambda t, c: (t, 0)),
            pl.BlockSpec((1, IN_FEATS, 128), lambda t, c: (c, 0, 0)),
            pl.BlockSpec((1, 128, 8), lambda t, c: (c, 0, 0)),
        ],
        out_specs=[
            pl.BlockSpec((1, BN, 128), lambda t, c: (c, t, 0)),
            pl.BlockSpec((BN, 4), lambda t, c: (t, 0)),
            pl.BlockSpec((BN, 4), lambda t, c: (t, 0)),
        ],
        out_shape=[
            jax.ShapeDtypeStruct((8, NPAD, 128), jnp.float32),
            jax.ShapeDtypeStruct((N, 4), jnp.float32),
            jax.ShapeDtypeStruct((N, 4), jnp.float32),
        ],
    )(features, W1r, P1)


def _k2(elr_flat, src_e, dst_e):
    @functools.partial(
        pl.kernel,
        out_type=jax.ShapeDtypeStruct((HEADS * E,), jnp.float32),
        mesh=_MESH,
        compiler_params=_CP,
        scratch_types=[
            pltpu.VMEM((4 * N,), jnp.float32),
            pltpu.VMEM((ET1,), jnp.int32),
            pltpu.VMEM((ET1,), jnp.int32),
            pltpu.VMEM((2 * EW,), jnp.float32),
        ],
    )
    def k(elr_hbm, src_hbm, dst_hbm, ee_hbm, elr_v, src_v, dst_v, eeb_v):
        cid = jax.lax.axis_index("c")
        sid = jax.lax.axis_index("s")
        ebase = sid * ET1

        pltpu.sync_copy(
            elr_hbm.at[pl.ds(pl.multiple_of(cid * (4 * N), 8), 4 * N)],
            elr_v)
        pltpu.sync_copy(src_hbm.at[pl.ds(ebase, ET1)], src_v)
        pltpu.sync_copy(dst_hbm.at[pl.ds(ebase, ET1)], dst_v)

        @pl.loop(0, NW1)
        def _(j):
            @pl.loop(0, EW, step=L)
            def _(k):
                s16 = src_v[pl.ds(j * EW + k, L)] * 4
                d16 = dst_v[pl.ds(j * EW + k, L)] * 4
                for h in range(2):
                    el = plsc.load_gather(elr_v, [s16 + (2 * h)])
                    er = plsc.load_gather(elr_v, [d16 + (2 * h + 1)])
                    x = el + er
                    x = jnp.where(x > 0, x, 0.2 * x)
                    eeb_v[pl.ds(h * EW + k, L)] = jnp.exp(x)

            for h in range(2):
                off = pl.multiple_of((2 * cid + h) * E + ebase + j * EW, 8)
                pltpu.sync_copy(eeb_v.at[pl.ds(h * EW, EW)],
                                ee_hbm.at[pl.ds(off, EW)])

    return k(elr_flat, src_e, dst_e)


def _k3(feat1, ee1, src_e, dst_e):
    ESEG = 2000

    @functools.partial(
        pl.kernel,
        out_type=[
            jax.ShapeDtypeStruct((8, NPAD, 128), jnp.float32),
            jax.ShapeDtypeStruct((HEADS * NPAD,), jnp.float32),
        ],
        mesh=_MESH,
        scratch_types=[
            pltpu.VMEM((ET1,), jnp.int32),
            pltpu.VMEM((ET1,), jnp.int32),
            pltpu.VMEM((MW,), jnp.int32),
            pltpu.VMEM((MW,), jnp.int32),
            pltpu.VMEM((ESEG,), jnp.float32),
            pltpu.VMEM((MW, 128), jnp.float32),
            pltpu.VMEM((MW, 128), jnp.float32),
            pltpu.VMEM((16, 128), jnp.float32),
            pltpu.VMEM((TPT,), jnp.float32),
            pltpu.VMEM((TPT,), jnp.float32),
            pltpu.VMEM_SHARED((NPAD, 128), jnp.float32),
            pltpu.VMEM_SHARED((NPAD,), jnp.float32),
            pltpu.SemaphoreType.DMA,
            pltpu.SemaphoreType.DMA,
        ],
    )
    def k(feat_hbm, ee_hbm, src_hbm, dst_hbm, out_hbm, den_hbm,
          src_v, dst_v, sidx0_v, sidx1_v, ees_v, rows0_v, rows1_v,
          zb_v, zb1_v, denw_v, acc_sh, den_sh, sem0, sem1):
        cid = jax.lax.axis_index("c")
        sid = jax.lax.axis_index("s")
        ebase = sid * ET1
        sidx = (sidx0_v, sidx1_v)
        rows = (rows0_v, rows1_v)
        sems = (sem0, sem1)

        def mo8(x):
            return x if isinstance(x, int) else pl.multiple_of(x, 8)

        @pl.loop(0, 16)
        def _(r):
            @pl.loop(0, 128, step=L)
            def _(d):
                zb_v[r, pl.ds(d, L)] = jnp.zeros((L,), jnp.float32)

        @pl.loop(0, TPT, step=L)
        def _(i):
            zb1_v[pl.ds(i, L)] = jnp.zeros((L,), jnp.float32)

        pltpu.sync_copy(src_hbm.at[pl.ds(ebase, ET1)], src_v)
        pltpu.sync_copy(dst_hbm.at[pl.ds(ebase, ET1)], dst_v)

        for i in range(4):
            ch = cid * 4 + i
            hg = ch // 2
            first_of_head = (i % 2 == 0)

            def seg_load(w):
                eoff = pl.multiple_of(
                    hg * E + ebase + (w // (ESEG // MW)) * ESEG, 8)
                pltpu.sync_copy(ee_hbm.at[pl.ds(eoff, ESEG)], ees_v)

            def prep(w, b):
                @pl.loop(0, MW, step=L)
                def _(k):
                    sidx[b][pl.ds(k, L)] = dst_v[pl.ds(w * MW + k, L)]

            @pl.loop(0, TPT // 16)
            def _(q):
                pltpu.sync_copy(zb_v, acc_sh.at[pl.ds(sid * TPT + q * 16,
                                                      16)])
            if first_of_head:
                pltpu.sync_copy(zb1_v, den_sh.at[pl.ds(sid * TPT, TPT)])
            plsc.subcore_barrier()

            prep(0, 0)
            pltpu.async_copy(
                feat_hbm.at[ch].at[src_v.at[pl.ds(0, MW)]],
                rows[0], sems[0])

            def wbody(w, par, last):
                npar = 1 - par
                if isinstance(w, int):
                    if w % (ESEG // MW) == 0:
                        seg_load(w)
                else:
                    @pl.when(w % (ESEG // MW) == 0)
                    def _():
                        seg_load(w)

                goff = mo8(w * MW)
                pltpu.make_async_copy(
                    feat_hbm.at[ch].at[src_v.at[pl.ds(goff, MW)]],
                    rows[par], sems[par]).wait()

                if not last:
                    prep(w + 1, npar)
                    ngoff = mo8((w + 1) * MW)
                    pltpu.async_copy(
                        feat_hbm.at[ch].at[src_v.at[pl.ds(ngoff, MW)]],
                        rows[npar], sems[npar])

                soff = mo8((w % (ESEG // MW)) * MW)

                @pl.loop(0, MW, step=L)
                def _(e):
                    ee16 = ees_v[pl.ds(soff + e, L)]
                    for jj in range(L):
                        sc = ee16[jj]
                        for d in range(0, 128, L):
                            rows[par][e + jj, pl.ds(d, L)] = (
                                rows[par][e + jj, pl.ds(d, L)] * sc)

                pltpu.sync_copy(rows[par], acc_sh.at[sidx[par]],
                                add=True)
                if first_of_head:
                    pltpu.sync_copy(ees_v.at[pl.ds(soff, MW)],
                                    den_sh.at[sidx[par]], add=True)

            @pl.loop(0, NMW // 2)
            def _(u):
                wbody(u * 2, 0, False)
                wbody(u * 2 + 1, 1, False)

            wbody(NMW - 1, 0, True)

            plsc.subcore_barrier()
            pltpu.sync_copy(acc_sh.at[pl.ds(sid * TPT, TPT)],
                            out_hbm.at[ch, pl.ds(sid * TPT, TPT)])
            if first_of_head:
                doff = pl.multiple_of(hg * NPAD + sid * TPT, 8)
                pltpu.sync_copy(den_sh.at[pl.ds(sid * TPT, TPT)], denw_v)
                pltpu.sync_copy(denw_v, den_hbm.at[pl.ds(doff, TPT)])
            plsc.subcore_barrier()

    return k(feat1, ee1, src_e, dst_e)


def _k4_body(msum_ref, den_ref, b1_ref, w2_ref, a2_ref, feat2_ref,
             elr2_ref):
    c = pl.program_id(1)
    dh = jnp.zeros((BN, 1), jnp.float32)
    for h in range(4):
        dh += jnp.where(c // 2 == h, den_ref[:, h:h + 1], 0.0)
    x = msum_ref[0] / (dh + 1e-9) + b1_ref[0]
    x = jnp.where(x > 0, x, jnp.exp(x) - 1.0)
    xw = jnp.dot(x, w2_ref[0], preferred_element_type=jnp.float32, precision=jax.lax.Precision.HIGHEST)

    @pl.when(c == 0)
    def _():
        feat2_ref[...] = xw

    @pl.when(c != 0)
    def _():
        feat2_ref[...] += xw

    @pl.when(c == 7)
    def _():
        elr2_ref[...] = jnp.dot(feat2_ref[...], a2_ref[...],
                                preferred_element_type=jnp.float32, precision=jax.lax.Precision.HIGHEST)


def _k4(msum1, den1t, b1r, W2r, A2):
    return pl.pallas_call(
        _k4_body,
        grid=(NT, 8),
        in_specs=[
            pl.BlockSpec((1, BN, 128), lambda t, c: (c, t, 0)),
            pl.BlockSpec((BN, 4), lambda t, c: (t, 0)),
            pl.BlockSpec((1, 1, 128), lambda t, c: (c, 0, 0)),
            pl.BlockSpec((1, 128, 256), lambda t, c: (c, 0, 0)),
            pl.BlockSpec((256, 2), lambda t, c: (0, 0)),
        ],
        out_specs=[
            pl.BlockSpec((BN, 256), lambda t, c: (t, 0)),
            pl.BlockSpec((BN, 2), lambda t, c: (t, 0)),
        ],
        out_shape=[
            jax.ShapeDtypeStruct((N, 256), jnp.float32),
            jax.ShapeDtypeStruct((N, 2), jnp.float32),
        ],
    )(msum1, den1t, b1r, W2r, A2)


def _k5(elr2_flat, src_e, dst_e):
    @functools.partial(
        pl.kernel,
        out_type=jax.ShapeDtypeStruct((E,), jnp.float32),
        mesh=_MESH,
        compiler_params=_CP,
        scratch_types=[
            pltpu.VMEM((2 * N,), jnp.float32),
            pltpu.VMEM((ET1,), jnp.int32),
            pltpu.VMEM((ET1,), jnp.int32),
            pltpu.VMEM((EW,), jnp.float32),
        ],
    )
    def k(elr_hbm, src_hbm, dst_hbm, ee_hbm, elr_v, src_v, dst_v, eeb_v):
        sid = jax.lax.axis_index("s")
        ebase = sid * ET1

        pltpu.sync_copy(elr_hbm, elr_v)
        pltpu.sync_copy(src_hbm.at[pl.ds(ebase, ET1)], src_v)
        pltpu.sync_copy(dst_hbm.at[pl.ds(ebase, ET1)], dst_v)

        @pl.loop(0, NW1)
        def _(j):
            @pl.loop(0, EW, step=L)
            def _(k):
                s16 = src_v[pl.ds(j * EW + k, L)] * 2
                d16 = dst_v[pl.ds(j * EW + k, L)] * 2
                el = plsc.load_gather(elr_v, [s16])
                er = plsc.load_gather(elr_v, [d16 + 1])
                x = el + er
                x = jnp.where(x > 0, x, 0.2 * x)
                eeb_v[pl.ds(k, L)] = jnp.exp(x)

            boff = pl.multiple_of(ebase + j * EW, 8)
            pltpu.sync_copy(eeb_v, ee_hbm.at[pl.ds(boff, EW)])

    return k(elr2_flat, src_e, dst_e)


def _k6(feat2c, ee2, src_e, dst_e):
    @functools.partial(
        pl.kernel,
        out_type=[
            jax.ShapeDtypeStruct((2, NPAD, 128), jnp.float32),
            jax.ShapeDtypeStruct((NPAD,), jnp.float32),
        ],
        mesh=_MESH,
        scratch_types=[
            pltpu.VMEM((ET1,), jnp.int32),
            pltpu.VMEM((ET1,), jnp.int32),
            pltpu.VMEM((ET1,), jnp.float32),
            pltpu.VMEM((MW,), jnp.int32),
            pltpu.VMEM((MW,), jnp.int32),
            pltpu.VMEM((MW,), jnp.float32),
            pltpu.VMEM((MW, 128), jnp.float32),
            pltpu.VMEM((MW, 128), jnp.float32),
            pltpu.VMEM((16, 128), jnp.float32),
            pltpu.VMEM((HPT,), jnp.float32),
            pltpu.VMEM((HPT,), jnp.float32),
            pltpu.VMEM_SHARED((HALF, 128), jnp.float32),
            pltpu.VMEM_SHARED((HALF,), jnp.float32),
            pltpu.SemaphoreType.DMA,
            pltpu.SemaphoreType.DMA,
        ],
    )
    def k(feat_hbm, ee_hbm, src_hbm, dst_hbm, out_hbm, den_hbm,
          src_v, dst_v, ee_v, sidx0_v, sidx1_v, em_v, rows0_v, rows1_v,
          zb_v, zb1_v, denw_v, acc_sh, den_sh, sem0, sem1):
        cid = jax.lax.axis_index("c")
        sid = jax.lax.axis_index("s")
        ebase = sid * ET1
        sidx = (sidx0_v, sidx1_v)
        rows = (rows0_v, rows1_v)
        sems = (sem0, sem1)

        def mo8(x):
            return x if isinstance(x, int) else pl.multiple_of(x, 8)

        @pl.loop(0, 16)
        def _(r):
            @pl.loop(0, 128, step=L)
            def _(d):
                zb_v[r, pl.ds(d, L)] = jnp.zeros((L,), jnp.float32)

        @pl.loop(0, HPT, step=L)
        def _(i):
            zb1_v[pl.ds(i, L)] = jnp.zeros((L,), jnp.float32)

        pltpu.sync_copy(src_hbm.at[pl.ds(ebase, ET1)], src_v)
        pltpu.sync_copy(dst_hbm.at[pl.ds(ebase, ET1)], dst_v)
        pltpu.sync_copy(ee_hbm.at[pl.ds(ebase, ET1)], ee_v)

        for p in range(2):
            base = p * HALF

            @pl.loop(0, HPT // 16)
            def _(q):
                pltpu.sync_copy(zb_v, acc_sh.at[pl.ds(sid * HPT + q * 16,
                                                      16)])
            pltpu.sync_copy(zb1_v, den_sh.at[pl.ds(sid * HPT, HPT)])
            plsc.subcore_barrier()

            def mkidx(w, b):
                @pl.loop(0, MW, step=L)
                def _(k):
                    d16 = dst_v[pl.ds(w * MW + k, L)]
                    din = d16 - base
                    ok = (din >= 0) & (din < HALF)
                    sidx[b][pl.ds(k, L)] = jnp.where(ok, din, d16 & 1023)
                    ee16 = ee_v[pl.ds(w * MW + k, L)]
                    em_v[pl.ds(k, L)] = jnp.where(ok, ee16, 0.0)

            mkidx(0, 0)
            pltpu.async_copy(
                feat_hbm.at[cid].at[src_v.at[pl.ds(0, MW)]],
                rows[0], sems[0])

            def wbody(w, par, last):
                npar = 1 - par
                goff = mo8(w * MW)
                pltpu.make_async_copy(
                    feat_hbm.at[cid].at[src_v.at[pl.ds(goff, MW)]],
                    rows[par], sems[par]).wait()

                @pl.when(cid == p)
                def _():
                    pltpu.sync_copy(em_v, den_sh.at[sidx[par]],
                                    add=True)

                if not last:
                    mkidx(w + 1, npar)
                    ngoff = mo8((w + 1) * MW)
                    pltpu.async_copy(
                        feat_hbm.at[cid].at[src_v.at[pl.ds(ngoff, MW)]],
                        rows[npar], sems[npar])

                @pl.loop(0, MW, step=L)
                def _(e):
                    d16 = dst_v[pl.ds(w * MW + e, L)]
                    din = d16 - base
                    ok = (din >= 0) & (din < HALF)
                    ee16 = jnp.where(ok, ee_v[pl.ds(w * MW + e, L)], 0.0)
                    for jj in range(L):
                        sc = ee16[jj]
                        for d in range(0, 128, L):
                            rows[par][e + jj, pl.ds(d, L)] = (
                                rows[par][e + jj, pl.ds(d, L)] * sc)

                pltpu.sync_copy(rows[par], acc_sh.at[sidx[par]],
                                add=True)

            @pl.loop(0, NMW // 2)
            def _(u):
                wbody(u * 2, 0, False)
                wbody(u * 2 + 1, 1, False)

            wbody(NMW - 1, 0, True)

            plsc.subcore_barrier()
            pltpu.sync_copy(
                acc_sh.at[pl.ds(sid * HPT, HPT)],
                out_hbm.at[cid, pl.ds(base + sid * HPT, HPT)])

            @pl.when(cid == p)
            def _():
                pltpu.sync_copy(den_sh.at[pl.ds(sid * HPT, HPT)], denw_v)
                pltpu.sync_copy(denw_v,
                                den_hbm.at[pl.ds(base + sid * HPT, HPT)])
            plsc.subcore_barrier()

    return k(feat2c, ee2, src_e, dst_e)


def _k7_body(msum_ref, den_ref, b2_ref, out_ref):
    d = den_ref[...] + 1e-9
    x = jnp.concatenate([msum_ref[0], msum_ref[1]], axis=1)
    out_ref[...] = x / d + b2_ref[...]


def _k7(msum2, den2n, b2r):
    return pl.pallas_call(
        _k7_body,
        grid=(NT,),
        in_specs=[
            pl.BlockSpec((2, BN, 128), lambda t: (0, t, 0)),
            pl.BlockSpec((BN, 1), lambda t: (t, 0)),
            pl.BlockSpec((1, 256), lambda t: (0, 0)),
        ],
        out_specs=pl.BlockSpec((BN, 256), lambda t: (t, 0)),
        out_shape=jax.ShapeDtypeStruct((N, 256), jnp.float32),
    )(msum2, den2n, b2r)


def kernel(features, edge_index, W1, al1, ar1, b1, W2, al2, ar2, b2):
    f32 = jnp.float32
    W1r = W1.reshape(IN_FEATS, 8, 128).transpose(1, 0, 2)
    W2r = W2.reshape(8, 128, 256)
    al_h = al1.reshape(HEADS, 2, 128)
    ar_h = ar1.reshape(HEADS, 2, 128)
    P1 = jnp.zeros((8, 128, 8), f32)
    for c in range(8):
        h, q = c // 2, c % 2
        P1 = P1.at[c, :, 2 * h].set(al_h[h, q])
        P1 = P1.at[c, :, 2 * h + 1].set(ar_h[h, q])
    A2 = jnp.stack([al2[0], ar2[0]], axis=1)
    b1r = b1.reshape(8, 1, 128)
    b2r = b2.reshape(1, 256)

    src_e = edge_index[0]
    dst_e = edge_index[1]

    feat1, elr_a, elr_b = _k1(features, W1r, P1)
    elr_flat = jnp.stack([elr_a, elr_b]).reshape(-1)
    ee1 = _k2(elr_flat, src_e, dst_e)
    msum1, den1 = _k3(feat1, ee1, src_e, dst_e)
    den1t = den1.reshape(HEADS, NPAD)[:, :N].T

    feat2, elr2 = _k4(msum1, den1t, b1r, W2r, A2)
    feat2c = feat2.reshape(N, 2, 128).transpose(1, 0, 2)
    ee2 = _k5(elr2.reshape(-1), src_e, dst_e)
    msum2, den2 = _k6(feat2c, ee2, src_e, dst_e)
    den2n = den2[:N].reshape(N, 1)

    return _k7(msum2, den2n, b2r).reshape(N, 1, 256)


# speedup vs baseline: 1.1848x; 1.1848x over previous
import dataclasses
import functools

import jax
import jax.numpy as jnp
from jax.experimental import pallas as pl
from jax.experimental.pallas import tpu as pltpu
from jax.experimental.pallas import tpu_sc as plsc

N = 10000
E = 160000
IN_FEATS = 256
HEADS = 4

NC, NS, L = 2, 16, 16
NPAD = 10240
TPT = NPAD // NS
HALF = NPAD // 2
HPT = HALF // NS
BN = 1000
NT = N // BN

EW = 400
ET1 = E // NS
NW1 = ET1 // EW
MW = 80
NMW = ET1 // MW

_MESH = plsc.VectorSubcoreMesh(core_axis_name="c", subcore_axis_name="s")
_CP = pltpu.CompilerParams()
if "needs_layout_passes" in pltpu.CompilerParams.__dataclass_fields__:
    _CP = dataclasses.replace(_CP, needs_layout_passes=False)


def _k1_body(x_ref, w_ref, p_ref, feat_ref, elr0_ref, elr1_ref):
    c = pl.program_id(1)
    fc = jnp.dot(x_ref[...], w_ref[0], preferred_element_type=jnp.float32)
    feat_ref[0] = fc
    pe = jnp.dot(fc, p_ref[0], preferred_element_type=jnp.float32)

    @pl.when(c == 0)
    def _():
        elr0_ref[...] = pe[:, :4]
        elr1_ref[...] = pe[:, 4:]

    @pl.when(c != 0)
    def _():
        elr0_ref[...] += pe[:, :4]
        elr1_ref[...] += pe[:, 4:]


def _k1(features, W1r, P1):
    return pl.pallas_call(
        _k1_body,
        grid=(NT, 8),
        in_specs=[
            pl.BlockSpec((BN, IN_FEATS), lambda t, c: (t, 0)),
            pl.BlockSpec((1, IN_FEATS, 128), lambda t, c: (c, 0, 0)),
            pl.BlockSpec((1, 128, 8), lambda t, c: (c, 0, 0)),
        ],
        out_specs=[
            pl.BlockSpec((1, BN, 128), lambda t, c: (c, t, 0)),
            pl.BlockSpec((BN, 4), lambda t, c: (t, 0)),
            pl.BlockSpec((BN, 4), lambda t, c: (t, 0)),
        ],
        out_shape=[
            jax.ShapeDtypeStruct((8, NPAD, 128), jnp.float32),
            jax.ShapeDtypeStruct((N, 4), jnp.float32),
            jax.ShapeDtypeStruct((N, 4), jnp.float32),
        ],
    )(features, W1r, P1)


def _k2(elr_flat, src_e, dst_e):
    @functools.partial(
        pl.kernel,
        out_type=jax.ShapeDtypeStruct((HEADS * E,), jnp.float32),
        mesh=_MESH,
        compiler_params=_CP,
        scratch_types=[
            pltpu.VMEM((4 * N,), jnp.float32),
            pltpu.VMEM((ET1,), jnp.int32),
            pltpu.VMEM((ET1,), jnp.int32),
            pltpu.VMEM((2 * EW,), jnp.float32),
        ],
    )
    def k(elr_hbm, src_hbm, dst_hbm, ee_hbm, elr_v, src_v, dst_v, eeb_v):
        cid = jax.lax.axis_index("c")
        sid = jax.lax.axis_index("s")
        ebase = sid * ET1

        pltpu.sync_copy(
            elr_hbm.at[pl.ds(pl.multiple_of(cid * (4 * N), 8), 4 * N)],
            elr_v)
        pltpu.sync_copy(src_hbm.at[pl.ds(ebase, ET1)], src_v)
        pltpu.sync_copy(dst_hbm.at[pl.ds(ebase, ET1)], dst_v)

        @pl.loop(0, NW1)
        def _(j):
            @pl.loop(0, EW, step=L)
            def _(k):
                s16 = src_v[pl.ds(j * EW + k, L)] * 4
                d16 = dst_v[pl.ds(j * EW + k, L)] * 4
                for h in range(2):
                    el = plsc.load_gather(elr_v, [s16 + (2 * h)])
                    er = plsc.load_gather(elr_v, [d16 + (2 * h + 1)])
                    x = el + er
                    x = jnp.where(x > 0, x, 0.2 * x)
                    eeb_v[pl.ds(h * EW + k, L)] = jnp.exp(x)

            for h in range(2):
                off = pl.multiple_of((2 * cid + h) * E + ebase + j * EW, 8)
                pltpu.sync_copy(eeb_v.at[pl.ds(h * EW, EW)],
                                ee_hbm.at[pl.ds(off, EW)])

    return k(elr_flat, src_e, dst_e)


def _k3(feat1, ee1, src_e, dst_e):
    ESEG = 2000

    @functools.partial(
        pl.kernel,
        out_type=[
            jax.ShapeDtypeStruct((8, NPAD, 128), jnp.float32),
            jax.ShapeDtypeStruct((HEADS * NPAD,), jnp.float32),
        ],
        mesh=_MESH,
        scratch_types=[
            pltpu.VMEM((ET1,), jnp.int32),
            pltpu.VMEM((ET1,), jnp.int32),
            pltpu.VMEM((MW,), jnp.int32),
            pltpu.VMEM((MW,), jnp.int32),
            pltpu.VMEM((ESEG,), jnp.float32),
            pltpu.VMEM((MW, 128), jnp.float32),
            pltpu.VMEM((MW, 128), jnp.float32),
            pltpu.VMEM((16, 128), jnp.float32),
            pltpu.VMEM((TPT,), jnp.float32),
            pltpu.VMEM((TPT,), jnp.float32),
            pltpu.VMEM_SHARED((NPAD, 128), jnp.float32),
            pltpu.VMEM_SHARED((NPAD,), jnp.float32),
            pltpu.SemaphoreType.DMA,
            pltpu.SemaphoreType.DMA,
            pltpu.SemaphoreType.DMA,
            pltpu.SemaphoreType.DMA,
        ],
    )
    def k(feat_hbm, ee_hbm, src_hbm, dst_hbm, out_hbm, den_hbm,
          src_v, dst_v, sidx0_v, sidx1_v, ees_v, rows0_v, rows1_v,
          zb_v, zb1_v, denw_v, acc_sh, den_sh, sem0, sem1, ssem0, ssem1):
        cid = jax.lax.axis_index("c")
        sid = jax.lax.axis_index("s")
        ebase = sid * ET1
        sidx = (sidx0_v, sidx1_v)
        rows = (rows0_v, rows1_v)
        sems = (sem0, sem1)
        ssems = (ssem0, ssem1)

        def mo8(x):
            return x if isinstance(x, int) else pl.multiple_of(x, 8)

        @pl.loop(0, 16)
        def _(r):
            @pl.loop(0, 128, step=L)
            def _(d):
                zb_v[r, pl.ds(d, L)] = jnp.zeros((L,), jnp.float32)

        @pl.loop(0, TPT, step=L)
        def _(i):
            zb1_v[pl.ds(i, L)] = jnp.zeros((L,), jnp.float32)

        pltpu.sync_copy(src_hbm.at[pl.ds(ebase, ET1)], src_v)
        pltpu.sync_copy(dst_hbm.at[pl.ds(ebase, ET1)], dst_v)

        for i in range(4):
            ch = cid * 4 + i
            hg = ch // 2
            first_of_head = (i % 2 == 0)

            def seg_load(w):
                eoff = pl.multiple_of(
                    hg * E + ebase + (w // (ESEG // MW)) * ESEG, 8)
                pltpu.sync_copy(ee_hbm.at[pl.ds(eoff, ESEG)], ees_v)

            def prep(w, b):
                @pl.loop(0, MW, step=L)
                def _(k):
                    sidx[b][pl.ds(k, L)] = dst_v[pl.ds(w * MW + k, L)]

            @pl.loop(0, TPT // 16)
            def _(q):
                pltpu.sync_copy(zb_v, acc_sh.at[pl.ds(sid * TPT + q * 16,
                                                      16)])
            if first_of_head:
                pltpu.sync_copy(zb1_v, den_sh.at[pl.ds(sid * TPT, TPT)])
            plsc.subcore_barrier()

            prep(0, 0)
            pltpu.async_copy(
                feat_hbm.at[ch].at[src_v.at[pl.ds(0, MW)]],
                rows[0], sems[0])

            def wbody(w, par, last):
                npar = 1 - par
                if isinstance(w, int):
                    if w % (ESEG // MW) == 0:
                        seg_load(w)
                else:
                    @pl.when(w % (ESEG // MW) == 0)
                    def _():
                        seg_load(w)

                goff = mo8(w * MW)
                pltpu.make_async_copy(
                    feat_hbm.at[ch].at[src_v.at[pl.ds(goff, MW)]],
                    rows[par], sems[par]).wait()

                if not last:
                    if isinstance(w, int):
                        if w >= 1:
                            pltpu.make_async_copy(
                                rows[npar], acc_sh.at[sidx[npar]],
                                ssems[npar]).wait()
                    else:
                        @pl.when(w >= 1)
                        def _():
                            pltpu.make_async_copy(
                                rows[npar], acc_sh.at[sidx[npar]],
                                ssems[npar]).wait()
                    prep(w + 1, npar)
                    ngoff = mo8((w + 1) * MW)
                    pltpu.async_copy(
                        feat_hbm.at[ch].at[src_v.at[pl.ds(ngoff, MW)]],
                        rows[npar], sems[npar])

                soff = mo8((w % (ESEG // MW)) * MW)

                @pl.loop(0, MW, step=L)
                def _(e):
                    ee16 = ees_v[pl.ds(soff + e, L)]
                    for jj in range(L):
                        sc = ee16[jj]
                        for d in range(0, 128, L):
                            rows[par][e + jj, pl.ds(d, L)] = (
                                rows[par][e + jj, pl.ds(d, L)] * sc)

                pltpu.async_copy(rows[par], acc_sh.at[sidx[par]],
                                 ssems[par], add=True)
                if first_of_head:
                    pltpu.sync_copy(ees_v.at[pl.ds(soff, MW)],
                                    den_sh.at[sidx[par]], add=True)

            @pl.loop(0, NMW // 2)
            def _(u):
                wbody(u * 2, 0, False)
                wbody(u * 2 + 1, 1, False)

            wbody(NMW - 1, 0, True)

            for b in range(2):
                pltpu.make_async_copy(rows[b], acc_sh.at[sidx[b]],
                                      ssems[b]).wait()

            plsc.subcore_barrier()
            pltpu.sync_copy(acc_sh.at[pl.ds(sid * TPT, TPT)],
                            out_hbm.at[ch, pl.ds(sid * TPT, TPT)])
            if first_of_head:
                doff = pl.multiple_of(hg * NPAD + sid * TPT, 8)
                pltpu.sync_copy(den_sh.at[pl.ds(sid * TPT, TPT)], denw_v)
                pltpu.sync_copy(denw_v, den_hbm.at[pl.ds(doff, TPT)])
            plsc.subcore_barrier()

    return k(feat1, ee1, src_e, dst_e)


def _k4_body(msum_ref, den_ref, b1_ref, w2_ref, a2_ref, feat2_ref,
             elr2_ref):
    c = pl.program_id(1)
    dh = jnp.zeros((BN, 1), jnp.float32)
    for h in range(4):
        dh += jnp.where(c // 2 == h, den_ref[:, h:h + 1], 0.0)
    x = msum_ref[0] / (dh + 1e-9) + b1_ref[0]
    x = jnp.where(x > 0, x, jnp.exp(x) - 1.0)
    xw = jnp.dot(x, w2_ref[0], preferred_element_type=jnp.float32)

    @pl.when(c == 0)
    def _():
        feat2_ref[...] = xw

    @pl.when(c != 0)
    def _():
        feat2_ref[...] += xw

    @pl.when(c == 7)
    def _():
        elr2_ref[...] = jnp.dot(feat2_ref[...], a2_ref[...],
                                preferred_element_type=jnp.float32)


def _k4(msum1, den1t, b1r, W2r, A2):
    return pl.pallas_call(
        _k4_body,
        grid=(NT, 8),
        in_specs=[
            pl.BlockSpec((1, BN, 128), lambda t, c: (c, t, 0)),
            pl.BlockSpec((BN, 4), lambda t, c: (t, 0)),
            pl.BlockSpec((1, 1, 128), lambda t, c: (c, 0, 0)),
            pl.BlockSpec((1, 128, 256), lambda t, c: (c, 0, 0)),
            pl.BlockSpec((256, 2), lambda t, c: (0, 0)),
        ],
        out_specs=[
            pl.BlockSpec((BN, 256), lambda t, c: (t, 0)),
            pl.BlockSpec((BN, 2), lambda t, c: (t, 0)),
        ],
        out_shape=[
            jax.ShapeDtypeStruct((N, 256), jnp.float32),
            jax.ShapeDtypeStruct((N, 2), jnp.float32),
        ],
    )(msum1, den1t, b1r, W2r, A2)


def _k5(elr2_flat, src_e, dst_e):
    @functools.partial(
        pl.kernel,
        out_type=jax.ShapeDtypeStruct((E,), jnp.float32),
        mesh=_MESH,
        compiler_params=_CP,
        scratch_types=[
            pltpu.VMEM((2 * N,), jnp.float32),
            pltpu.VMEM((ET1,), jnp.int32),
            pltpu.VMEM((ET1,), jnp.int32),
            pltpu.VMEM((EW,), jnp.float32),
        ],
    )
    def k(elr_hbm, src_hbm, dst_hbm, ee_hbm, elr_v, src_v, dst_v, eeb_v):
        sid = jax.lax.axis_index("s")
        ebase = sid * ET1

        pltpu.sync_copy(elr_hbm, elr_v)
        pltpu.sync_copy(src_hbm.at[pl.ds(ebase, ET1)], src_v)
        pltpu.sync_copy(dst_hbm.at[pl.ds(ebase, ET1)], dst_v)

        @pl.loop(0, NW1)
        def _(j):
            @pl.loop(0, EW, step=L)
            def _(k):
                s16 = src_v[pl.ds(j * EW + k, L)] * 2
                d16 = dst_v[pl.ds(j * EW + k, L)] * 2
                el = plsc.load_gather(elr_v, [s16])
                er = plsc.load_gather(elr_v, [d16 + 1])
                x = el + er
                x = jnp.where(x > 0, x, 0.2 * x)
                eeb_v[pl.ds(k, L)] = jnp.exp(x)

            boff = pl.multiple_of(ebase + j * EW, 8)
            pltpu.sync_copy(eeb_v, ee_hbm.at[pl.ds(boff, EW)])

    return k(elr2_flat, src_e, dst_e)


def _k6(feat2c, ee2, src_e, dst_e):
    @functools.partial(
        pl.kernel,
        out_type=[
            jax.ShapeDtypeStruct((2, NPAD, 128), jnp.float32),
            jax.ShapeDtypeStruct((NPAD,), jnp.float32),
        ],
        mesh=_MESH,
        scratch_types=[
            pltpu.VMEM((ET1,), jnp.int32),
            pltpu.VMEM((ET1,), jnp.int32),
            pltpu.VMEM((ET1,), jnp.float32),
            pltpu.VMEM((MW,), jnp.int32),
            pltpu.VMEM((MW,), jnp.int32),
            pltpu.VMEM((MW,), jnp.float32),
            pltpu.VMEM((MW, 128), jnp.float32),
            pltpu.VMEM((MW, 128), jnp.float32),
            pltpu.VMEM((16, 128), jnp.float32),
            pltpu.VMEM((HPT,), jnp.float32),
            pltpu.VMEM((HPT,), jnp.float32),
            pltpu.VMEM_SHARED((HALF, 128), jnp.float32),
            pltpu.VMEM_SHARED((HALF,), jnp.float32),
            pltpu.SemaphoreType.DMA,
            pltpu.SemaphoreType.DMA,
        ],
    )
    def k(feat_hbm, ee_hbm, src_hbm, dst_hbm, out_hbm, den_hbm,
          src_v, dst_v, ee_v, sidx0_v, sidx1_v, em_v, rows0_v, rows1_v,
          zb_v, zb1_v, denw_v, acc_sh, den_sh, sem0, sem1):
        cid = jax.lax.axis_index("c")
        sid = jax.lax.axis_index("s")
        ebase = sid * ET1
        sidx = (sidx0_v, sidx1_v)
        rows = (rows0_v, rows1_v)
        sems = (sem0, sem1)

        def mo8(x):
            return x if isinstance(x, int) else pl.multiple_of(x, 8)

        @pl.loop(0, 16)
        def _(r):
            @pl.loop(0, 128, step=L)
            def _(d):
                zb_v[r, pl.ds(d, L)] = jnp.zeros((L,), jnp.float32)

        @pl.loop(0, HPT, step=L)
        def _(i):
            zb1_v[pl.ds(i, L)] = jnp.zeros((L,), jnp.float32)

        pltpu.sync_copy(src_hbm.at[pl.ds(ebase, ET1)], src_v)
        pltpu.sync_copy(dst_hbm.at[pl.ds(ebase, ET1)], dst_v)
        pltpu.sync_copy(ee_hbm.at[pl.ds(ebase, ET1)], ee_v)

        for p in range(2):
            base = p * HALF

            @pl.loop(0, HPT // 16)
            def _(q):
                pltpu.sync_copy(zb_v, acc_sh.at[pl.ds(sid * HPT + q * 16,
                                                      16)])
            pltpu.sync_copy(zb1_v, den_sh.at[pl.ds(sid * HPT, HPT)])
            plsc.subcore_barrier()

            def mkidx(w, b):
                @pl.loop(0, MW, step=L)
                def _(k):
                    d16 = dst_v[pl.ds(w * MW + k, L)]
                    din = d16 - base
                    ok = (din >= 0) & (din < HALF)
                    sidx[b][pl.ds(k, L)] = jnp.where(ok, din, d16 & 1023)
                    ee16 = ee_v[pl.ds(w * MW + k, L)]
                    em_v[pl.ds(k, L)] = jnp.where(ok, ee16, 0.0)

            mkidx(0, 0)
            pltpu.async_copy(
                feat_hbm.at[cid].at[src_v.at[pl.ds(0, MW)]],
                rows[0], sems[0])

            def wbody(w, par, last):
                npar = 1 - par
                goff = mo8(w * MW)
                pltpu.make_async_copy(
                    feat_hbm.at[cid].at[src_v.at[pl.ds(goff, MW)]],
                    rows[par], sems[par]).wait()

                @pl.when(cid == p)
                def _():
                    pltpu.sync_copy(em_v, den_sh.at[sidx[par]],
                                    add=True)

                if not last:
                    mkidx(w + 1, npar)
                    ngoff = mo8((w + 1) * MW)
                    pltpu.async_copy(
                        feat_hbm.at[cid].at[src_v.at[pl.ds(ngoff, MW)]],
                        rows[npar], sems[npar])

                @pl.loop(0, MW, step=L)
                def _(e):
                    d16 = dst_v[pl.ds(w * MW + e, L)]
                    din = d16 - base
                    ok = (din >= 0) & (din < HALF)
                    ee16 = jnp.where(ok, ee_v[pl.ds(w * MW + e, L)], 0.0)
                    for jj in range(L):
                        sc = ee16[jj]
                        for d in range(0, 128, L):
                            rows[par][e + jj, pl.ds(d, L)] = (
                                rows[par][e + jj, pl.ds(d, L)] * sc)

                pltpu.sync_copy(rows[par], acc_sh.at[sidx[par]],
                                add=True)

            @pl.loop(0, NMW // 2)
            def _(u):
                wbody(u * 2, 0, False)
                wbody(u * 2 + 1, 1, False)

            wbody(NMW - 1, 0, True)

            plsc.subcore_barrier()
            pltpu.sync_copy(
                acc_sh.at[pl.ds(sid * HPT, HPT)],
                out_hbm.at[cid, pl.ds(base + sid * HPT, HPT)])

            @pl.when(cid == p)
            def _():
                pltpu.sync_copy(den_sh.at[pl.ds(sid * HPT, HPT)], denw_v)
                pltpu.sync_copy(denw_v,
                                den_hbm.at[pl.ds(base + sid * HPT, HPT)])
            plsc.subcore_barrier()

    return k(feat2c, ee2, src_e, dst_e)


def _k7_body(msum_ref, den_ref, b2_ref, out_ref):
    d = den_ref[...] + 1e-9
    x = jnp.concatenate([msum_ref[0], msum_ref[1]], axis=1)
    out_ref[...] = x / d + b2_ref[...]


def _k7(msum2, den2n, b2r):
    return pl.pallas_call(
        _k7_body,
        grid=(NT,),
        in_specs=[
            pl.BlockSpec((2, BN, 128), lambda t: (0, t, 0)),
            pl.BlockSpec((BN, 1), lambda t: (t, 0)),
            pl.BlockSpec((1, 256), lambda t: (0, 0)),
        ],
        out_specs=pl.BlockSpec((BN, 256), lambda t: (t, 0)),
        out_shape=jax.ShapeDtypeStruct((N, 256), jnp.float32),
    )(msum2, den2n, b2r)


def kernel(features, edge_index, W1, al1, ar1, b1, W2, al2, ar2, b2):
    f32 = jnp.float32
    W1r = W1.reshape(IN_FEATS, 8, 128).transpose(1, 0, 2)
    W2r = W2.reshape(8, 128, 256)
    al_h = al1.reshape(HEADS, 2, 128)
    ar_h = ar1.reshape(HEADS, 2, 128)
    P1 = jnp.zeros((8, 128, 8), f32)
    for c in range(8):
        h, q = c // 2, c % 2
        P1 = P1.at[c, :, 2 * h].set(al_h[h, q])
        P1 = P1.at[c, :, 2 * h + 1].set(ar_h[h, q])
    A2 = jnp.stack([al2[0], ar2[0]], axis=1)
    b1r = b1.reshape(8, 1, 128)
    b2r = b2.reshape(1, 256)

    src_e = edge_index[0]
    dst_e = edge_index[1]

    feat1, elr_a, elr_b = _k1(features, W1r, P1)
    elr_flat = jnp.stack([elr_a, elr_b]).reshape(-1)
    ee1 = _k2(elr_flat, src_e, dst_e)
    msum1, den1 = _k3(feat1, ee1, src_e, dst_e)
    den1t = den1.reshape(HEADS, NPAD)[:, :N].T

    feat2, elr2 = _k4(msum1, den1t, b1r, W2r, A2)
    feat2c = feat2.reshape(N, 2, 128).transpose(1, 0, 2)
    ee2 = _k5(elr2.reshape(-1), src_e, dst_e)
    msum2, den2 = _k6(feat2c, ee2, src_e, dst_e)
    den2n = den2[:N].reshape(N, 1)

    return _k7(msum2, den2n, b2r).reshape(N, 1, 256)
